# baseline (speedup 1.0000x reference)
import jax
import jax.numpy as jnp
from jax.experimental import pallas as pl
from jax.experimental.pallas import tpu as pltpu
from jax.experimental.pallas import tpu_sc as plsc

_W = 128


def _sc_gather(token_table, idx_2d, n, h):
    mesh = plsc.VectorSubcoreMesh(core_axis_name="core", subcore_axis_name="subcore")

    @pl.kernel(
        out_type=jax.ShapeDtypeStruct((n, h), token_table.dtype),
        mesh=mesh,
        compiler_params=pltpu.CompilerParams(use_tc_tiling_on_sc=False),
    )
    def gather_kernel(tok_hbm, i_hbm, o_hbm):
        def body(i_vmem, o_vmem):
            pltpu.sync_copy(tok_hbm.at[i_vmem.at[0]], o_vmem)

        pltpu.emit_pipeline(
            body,
            grid=(n // _W,),
            in_specs=[pl.BlockSpec((1, _W), lambda i: (0, i))],
            out_specs=[pl.BlockSpec((_W, h), lambda i: (i, 0))],
            core_axis_name=("core", "subcore"),
            dimension_semantics=(pltpu.PARALLEL,),
        )(i_hbm, o_hbm)

    return gather_kernel(token_table, idx_2d)


def _tc_pos_add(tok3, pos3):
    s, b, h = tok3.shape

    def add_body(t_ref, p_ref, o_ref):
        o_ref[...] = t_ref[...] + p_ref[...]

    return pl.pallas_call(
        add_body,
        grid=(s,),
        in_specs=[
            pl.BlockSpec((1, b, h), lambda i: (i, 0, 0)),
            pl.BlockSpec((1, 1, h), lambda i: (i, 0, 0)),
        ],
        out_specs=pl.BlockSpec((1, b, h), lambda i: (i, 0, 0)),
        out_shape=jax.ShapeDtypeStruct((s, b, h), tok3.dtype),
    )(tok3, pos3)


def kernel(input_ids, token_table, position_table):
    batch, seq = input_ids.shape
    _, hidden = token_table.shape
    n = batch * seq

    idx = jnp.swapaxes(input_ids, 0, 1).astype(jnp.int32).reshape(1, n)

    tok = _sc_gather(token_table, idx, n, hidden)
    tok3 = tok.reshape(seq, batch, hidden)
    pos3 = position_table.reshape(seq, 1, hidden)
    return _tc_pos_add(tok3, pos3)


# speedup vs baseline: 1.1009x; 1.1009x over previous
import jax
import jax.numpy as jnp
from jax.experimental import pallas as pl
from jax.experimental.pallas import tpu as pltpu
from jax.experimental.pallas import tpu_sc as plsc

_W = 128


def _tc_prep(ids):
    b, s = ids.shape

    def body(i_ref, h_ref, p_ref):
        x = jnp.swapaxes(i_ref[...], 0, 1)
        h_ref[...] = x >> 1
        p_ref[...] = x & 1

    return pl.pallas_call(
        body,
        out_shape=(
            jax.ShapeDtypeStruct((s, b), jnp.int32),
            jax.ShapeDtypeStruct((s, b), jnp.int32),
        ),
    )(ids.astype(jnp.int32))


def _sc_gather_pairs(pairs, idx_half):
    s, b = idx_half.shape
    n = s * b
    mesh = plsc.VectorSubcoreMesh(core_axis_name="core", subcore_axis_name="subcore")
    wpr = b // _W

    @pl.kernel(
        out_type=jax.ShapeDtypeStruct((n, 128), pairs.dtype),
        mesh=mesh,
    )
    def gather_kernel(pairs_hbm, i_hbm, o_hbm):
        def body(i_vmem, o_vmem):
            pltpu.sync_copy(pairs_hbm.at[i_vmem.at[0]], o_vmem)

        pltpu.emit_pipeline(
            body,
            grid=(s, wpr),
            in_specs=[pl.BlockSpec((1, _W), lambda i, j: (i, j))],
            out_specs=[pl.BlockSpec((_W, 128), lambda i, j, _wpr=wpr: (i * _wpr + j, 0))],
            core_axis_name=("core", "subcore"),
            dimension_semantics=(pltpu.PARALLEL, pltpu.PARALLEL),
        )(i_hbm, o_hbm)

    return gather_kernel(pairs, idx_half)


def _tc_select_add(tmp3, par, pos):
    seq, batch, _ = tmp3.shape
    h = pos.shape[-1]
    sc = 8

    def body(t_ref, p_ref, e_ref, o_ref):
        x = t_ref[...]
        sh = x.shape[:2] + (h,)
        p3 = jax.lax.broadcast_in_dim(p_ref[...], sh, (0, 1))
        sel = jnp.where(p3 == 1, x[:, :, h:], x[:, :, :h])
        o_ref[...] = sel + e_ref[...][:, None, :]

    return pl.pallas_call(
        body,
        grid=(seq // sc,),
        in_specs=[
            pl.BlockSpec((sc, batch, 128), lambda i: (i, 0, 0)),
            pl.BlockSpec((sc, batch), lambda i: (i, 0)),
            pl.BlockSpec((sc, h), lambda i: (i, 0)),
        ],
        out_specs=pl.BlockSpec((sc, batch, h), lambda i: (i, 0, 0)),
        out_shape=jax.ShapeDtypeStruct((seq, batch, h), tmp3.dtype),
    )(tmp3, par, pos)


def kernel(input_ids, token_table, position_table):
    batch, seq = input_ids.shape
    vocab, hidden = token_table.shape

    idx_half, par = _tc_prep(input_ids)
    pairs = token_table.reshape(vocab // 2, 2 * hidden)
    tmp = _sc_gather_pairs(pairs, idx_half)
    tmp3 = tmp.reshape(seq, batch, 2 * hidden)
    return _tc_select_add(tmp3, par, position_table)


# speedup vs baseline: 1.7163x; 1.5590x over previous
import jax
import jax.numpy as jnp
from jax.experimental import pallas as pl
from jax.experimental.pallas import tpu as pltpu
from jax.experimental.pallas import tpu_sc as plsc

_W = 128
_PW = 2048
_OFF = 501760


def _tc_prep(ids):
    b, s = ids.shape

    def body(i_ref, h_ref, p_ref):
        x = jnp.swapaxes(i_ref[...], 0, 1)
        hi = (x >= _OFF).astype(jnp.int32)
        h_ref[...] = x - _OFF * hi
        p_ref[...] = hi

    return pl.pallas_call(
        body,
        out_shape=(
            jax.ShapeDtypeStruct((s, b), jnp.int32),
            jax.ShapeDtypeStruct((s, b), jnp.int32),
        ),
    )(ids.astype(jnp.int32))


def _tc_pairs(table_t):
    h, v = table_t.shape
    n_blocks = pl.cdiv(v, _PW)
    half_blocks = _OFF // _PW

    def body(a_ref, b_ref, o_ref):
        o_ref[:, :h] = jnp.swapaxes(a_ref[...], 0, 1)
        o_ref[:, h:] = jnp.swapaxes(b_ref[...], 0, 1)

    return pl.pallas_call(
        body,
        grid=(half_blocks,),
        in_specs=[
            pl.BlockSpec((h, _PW), lambda i: (0, i)),
            pl.BlockSpec(
                (h, _PW),
                lambda i, _hb=half_blocks, _nb=n_blocks: (
                    0, jnp.minimum(i + _hb, _nb - 1))),
        ],
        out_specs=pl.BlockSpec((_PW, 2 * h), lambda i: (i, 0)),
        out_shape=jax.ShapeDtypeStruct((_OFF, 2 * h), table_t.dtype),
    )(table_t, table_t)


def _sc_gather(pairs, row_idx):
    s, b = row_idx.shape
    n = s * b
    mesh = plsc.VectorSubcoreMesh(core_axis_name="core", subcore_axis_name="subcore")
    wpr = b // _W

    @pl.kernel(
        out_type=jax.ShapeDtypeStruct((n, 128), pairs.dtype),
        mesh=mesh,
    )
    def gather_kernel(pairs_hbm, i_hbm, o_hbm):
        def body(i_vmem, o_vmem):
            pltpu.sync_copy(pairs_hbm.at[i_vmem.at[0]], o_vmem)

        pltpu.emit_pipeline(
            body,
            grid=(s, wpr),
            in_specs=[pl.BlockSpec((1, _W), lambda i, j: (i, j))],
            out_specs=[pl.BlockSpec((_W, 128), lambda i, j, _wpr=wpr: (i * _wpr + j, 0))],
            core_axis_name=("core", "subcore"),
            dimension_semantics=(pltpu.PARALLEL, pltpu.PARALLEL),
        )(i_hbm, o_hbm)

    return gather_kernel(pairs, row_idx)


def _tc_select_add_t(tmp3, half, pos):
    seq, batch, _ = tmp3.shape
    h = pos.shape[-1]
    sc = 8

    def body(t_ref, p_ref, e_ref, o_ref):
        x = t_ref[...]
        p3 = jax.lax.broadcast_in_dim(p_ref[...], (sc, batch, h), (0, 1))
        sel = jnp.where(p3 == 1, x[:, :, h:], x[:, :, :h])
        selt = jnp.transpose(sel, (0, 2, 1))
        o_ref[...] = selt + jax.lax.broadcast_in_dim(e_ref[...], (sc, h, batch), (0, 1))

    return pl.pallas_call(
        body,
        grid=(seq // sc,),
        in_specs=[
            pl.BlockSpec((sc, batch, 128), lambda i: (i, 0, 0)),
            pl.BlockSpec((sc, batch), lambda i: (i, 0)),
            pl.BlockSpec((sc, h), lambda i: (i, 0)),
        ],
        out_specs=pl.BlockSpec((sc, h, batch), lambda i: (i, 0, 0)),
        out_shape=jax.ShapeDtypeStruct((seq, h, batch), tmp3.dtype),
    )(tmp3, half, pos)


def kernel(input_ids, token_table, position_table):
    batch, seq = input_ids.shape
    vocab, hidden = token_table.shape

    row_idx, half = _tc_prep(input_ids)
    table_t = jnp.swapaxes(token_table, 0, 1)
    pairs = _tc_pairs(table_t)
    tmp = _sc_gather(pairs, row_idx)
    tmp3 = tmp.reshape(seq, batch, 2 * hidden)
    out_t = _tc_select_add_t(tmp3, half, position_table)
    return jnp.swapaxes(out_t, 1, 2)
